# Initial kernel scaffold; baseline (speedup 1.0000x reference)
import jax
import jax.numpy as jnp
from jax.experimental import pallas as pl


def kernel(h, edge_index, coord, edge_attr, e_w1, e_b1, e_w2, e_b2, n_w1, n_b1, n_w2, n_b2, c_w1, c_b1, c_w2, ln_g, ln_b):
    raise NotImplementedError("write your pallas kernel here")


# speedup vs baseline: 3.9448x; 3.9448x over previous
import functools

import jax
import jax.numpy as jnp
from jax import lax
from jax.experimental import pallas as pl
from jax.experimental.pallas import tpu as pltpu
from jax.experimental.pallas import tpu_sc as plsc

N = 10000
E = 320000
D = 128
DE = 16
NC = 2
NS = 16
NW = NC * NS
EPW = E // NW
CH = 80
NCH = EPW // CH
GR = CH // 16
RPT = N // NS
TSW = 16

_mesh = plsc.VectorSubcoreMesh(core_axis_name="c", subcore_axis_name="s")
_f32 = jnp.float32


def _proj_body(h_ref, wr_ref, wc_ref, b1_ref, hr_ref, hc_ref):
    h = h_ref[...]
    hr_ref[...] = jnp.dot(h, wr_ref[...], preferred_element_type=_f32)
    hc_ref[...] = jnp.dot(h, wc_ref[...], preferred_element_type=_f32) + b1_ref[...]


def _stage_a(h, w_hr, w_hc, b1):
    bn = 2000
    grid = (N // bn,)
    return pl.pallas_call(
        _proj_body,
        grid=grid,
        in_specs=[
            pl.BlockSpec((bn, D), lambda i: (i, 0)),
            pl.BlockSpec((D, D), lambda i: (0, 0)),
            pl.BlockSpec((D, D), lambda i: (0, 0)),
            pl.BlockSpec((1, D), lambda i: (0, 0)),
        ],
        out_specs=[
            pl.BlockSpec((bn, D), lambda i: (i, 0)),
            pl.BlockSpec((bn, D), lambda i: (i, 0)),
        ],
        out_shape=[
            jax.ShapeDtypeStruct((N, D), _f32),
            jax.ShapeDtypeStruct((N, D), _f32),
        ],
    )(h, w_hr, w_hc, b1)


def _gather_body(hr_hbm, hc_hbm, row_hbm, col_hbm, coord_hbm,
                 pre_hbm, rad_hbm,
                 coord_v, idxr_v, idxc_v, rows_a, rows_b, rad_v, sem):
    cid = lax.axis_index("c")
    sid = lax.axis_index("s")
    wid = sid * NC + cid
    base = wid * EPW
    pltpu.sync_copy(coord_hbm, coord_v)

    def chunk(c, carry):
        off = base + c * CH
        pltpu.sync_copy(row_hbm.at[pl.ds(off, CH)], idxr_v)
        pltpu.sync_copy(col_hbm.at[pl.ds(off, CH)], idxc_v)
        pltpu.async_copy(hr_hbm.at[idxr_v], rows_a, sem).wait()
        pltpu.async_copy(hc_hbm.at[idxc_v], rows_b, sem).wait()

        def addrow(i, acc):
            for j in range(D // 16):
                sl = pl.ds(j * 16, 16)
                rows_a[i, sl] = rows_a[i, sl] + rows_b[i, sl]
            return acc
        lax.fori_loop(0, CH, addrow, 0)

        z = jnp.zeros((16,), jnp.int32)
        for g in range(GR):
            sl = pl.ds(g * 16, 16)
            rowv = idxr_v[sl]
            colv = idxc_v[sl]
            crx = plsc.load_gather(coord_v, [rowv, z])
            cry = plsc.load_gather(coord_v, [rowv, z + 1])
            crz = plsc.load_gather(coord_v, [rowv, z + 2])
            ccx = plsc.load_gather(coord_v, [colv, z])
            ccy = plsc.load_gather(coord_v, [colv, z + 1])
            ccz = plsc.load_gather(coord_v, [colv, z + 2])
            dx = crx - ccx
            dy = cry - ccy
            dz = crz - ccz
            rad_v[sl] = dx * dx + dy * dy + dz * dz

        pltpu.sync_copy(rows_a, pre_hbm.at[pl.ds(off, CH)])
        pltpu.sync_copy(rad_v, rad_hbm.at[pl.ds(off, CH)])
        return carry
    lax.fori_loop(0, NCH, chunk, 0)


def _stage_b(hr, hc, row, col, coordp):
    fn = pl.kernel(
        _gather_body,
        mesh=_mesh,
        compiler_params=pltpu.CompilerParams(needs_layout_passes=False, use_tc_tiling_on_sc=False),
        out_type=[
            jax.ShapeDtypeStruct((E, D), _f32),
            jax.ShapeDtypeStruct((E,), _f32),
        ],
        scratch_types=[
            pltpu.VMEM((N, 4), _f32),
            pltpu.VMEM((CH,), jnp.int32),
            pltpu.VMEM((CH,), jnp.int32),
            pltpu.VMEM((CH, D), _f32),
            pltpu.VMEM((CH, D), _f32),
            pltpu.VMEM((CH,), _f32),
            pltpu.SemaphoreType.DMA,
        ],
    )
    return fn(hr, hc, row, col, coordp)


def _edge_mlp_body(pre_ref, rad_ref, attr_ref, wrad_ref, wattr_ref,
                   w2_ref, b2_ref, cw1_ref, cb1_ref, cw2_ref,
                   ef_ref, coff_ref):
    x = pre_ref[...] + rad_ref[...] * wrad_ref[...]
    x = x + jnp.dot(attr_ref[...], wattr_ref[...], preferred_element_type=_f32)
    x = jnp.maximum(x, 0.0)
    ef = jnp.dot(x, w2_ref[...], preferred_element_type=_f32) + b2_ref[...]
    ef = jnp.maximum(ef, 0.0)
    y = jnp.dot(ef, cw1_ref[...], preferred_element_type=_f32) + cb1_ref[...]
    y = jnp.maximum(y, 0.0)
    coff_ref[...] = jnp.dot(y, cw2_ref[...], preferred_element_type=_f32)
    ef_ref[...] = ef


def _stage_c(pre, rad, attr, w_rad, w_attr, e_w2, e_b2, c_w1, c_b1, cw2p):
    be = 2000
    grid = (E // be,)
    return pl.pallas_call(
        _edge_mlp_body,
        grid=grid,
        in_specs=[
            pl.BlockSpec((be, D), lambda i: (i, 0)),
            pl.BlockSpec((be, 1), lambda i: (i, 0)),
            pl.BlockSpec((be, DE), lambda i: (i, 0)),
            pl.BlockSpec((1, D), lambda i: (0, 0)),
            pl.BlockSpec((DE, D), lambda i: (0, 0)),
            pl.BlockSpec((D, D), lambda i: (0, 0)),
            pl.BlockSpec((1, D), lambda i: (0, 0)),
            pl.BlockSpec((D, D), lambda i: (0, 0)),
            pl.BlockSpec((1, D), lambda i: (0, 0)),
            pl.BlockSpec((D, 8), lambda i: (0, 0)),
        ],
        out_specs=[
            pl.BlockSpec((be, D), lambda i: (i, 0)),
            pl.BlockSpec((be, 8), lambda i: (i, 0)),
        ],
        out_shape=[
            jax.ShapeDtypeStruct((E, D), _f32),
            jax.ShapeDtypeStruct((E, 8), _f32),
        ],
    )(pre, rad, attr, w_rad, w_attr, e_w2, e_b2, c_w1, c_b1, cw2p)


def _aggh_body(row_hbm, ef_hbm, z128_hbm, aggh_out,
               idxr_v, ef_v, aggh_sh):
    cid = lax.axis_index("c")
    sid = lax.axis_index("s")
    wid = sid * NC + cid
    base = wid * EPW

    rsl = pl.ds(sid * RPT, RPT)
    pltpu.sync_copy(z128_hbm.at[rsl], aggh_sh.at[rsl])
    plsc.subcore_barrier()

    def chunk(c, carry):
        off = base + c * CH
        pltpu.sync_copy(row_hbm.at[pl.ds(off, CH)], idxr_v)
        pltpu.sync_copy(ef_hbm.at[pl.ds(off, CH)], ef_v)
        pltpu.sync_copy(ef_v, aggh_sh.at[idxr_v], add=True)
        return carry
    lax.fori_loop(0, NCH, chunk, 0)

    plsc.subcore_barrier()
    pltpu.sync_copy(aggh_sh.at[rsl], aggh_out.at[cid, rsl])


def _stage_d1(row, ef, z128):
    fn = pl.kernel(
        _aggh_body,
        mesh=_mesh,
        compiler_params=pltpu.CompilerParams(needs_layout_passes=False, use_tc_tiling_on_sc=False),
        out_type=jax.ShapeDtypeStruct((NC, N, D), _f32),
        scratch_types=[
            pltpu.VMEM((CH,), jnp.int32),
            pltpu.VMEM((CH, D), _f32),
            pltpu.VMEM_SHARED((N, D), _f32),
        ],
    )
    return fn(row, ef, z128)


def _coordagg_body(row_hbm, col_hbm, coord_hbm, coff_hbm, zts_hbm,
                   tsc_out,
                   coord_v, idxr_v, idxc_v, coff_v, tc_v, tsc_sh):
    cid = lax.axis_index("c")
    sid = lax.axis_index("s")
    wid = sid * NC + cid
    base = wid * EPW
    pltpu.sync_copy(coord_hbm, coord_v)

    def ztc(i, acc):
        tc_v[i, pl.ds(0, TSW)] = jnp.zeros((TSW,), _f32)
        return acc
    lax.fori_loop(0, CH, ztc, 0)

    rsl = pl.ds(sid * RPT, RPT)
    pltpu.sync_copy(zts_hbm.at[rsl], tsc_sh.at[rsl])
    plsc.subcore_barrier()

    iota = lax.iota(jnp.int32, 16)
    ones = jnp.ones((16,), _f32)

    def chunk(c, carry):
        off = base + c * CH
        pltpu.sync_copy(row_hbm.at[pl.ds(off, CH)], idxr_v)
        pltpu.sync_copy(col_hbm.at[pl.ds(off, CH)], idxc_v)
        pltpu.sync_copy(coff_hbm.at[pl.ds(off, CH)], coff_v)

        z = jnp.zeros((16,), jnp.int32)
        for g in range(GR):
            sl = pl.ds(g * 16, 16)
            rowv = idxr_v[sl]
            colv = idxc_v[sl]
            eidx = iota + (g * 16)
            crx = plsc.load_gather(coord_v, [rowv, z])
            cry = plsc.load_gather(coord_v, [rowv, z + 1])
            crz = plsc.load_gather(coord_v, [rowv, z + 2])
            ccx = plsc.load_gather(coord_v, [colv, z])
            ccy = plsc.load_gather(coord_v, [colv, z + 1])
            ccz = plsc.load_gather(coord_v, [colv, z + 2])
            c0 = plsc.load_gather(coff_v, [eidx, z])
            c1 = plsc.load_gather(coff_v, [eidx, z + 1])
            c2 = plsc.load_gather(coff_v, [eidx, z + 2])
            dx = crx - ccx
            dy = cry - ccy
            dz = crz - ccz
            cx = cry * ccz - crz * ccy
            cy = crz * ccx - crx * ccz
            cz = crx * ccy - cry * ccx
            vx = dy * cz - dz * cy
            vy = dz * cx - dx * cz
            vz = dx * cy - dy * cx
            tx = dx * c0 + cx * c1 + vx * c2
            ty = dy * c0 + cy * c1 + vy * c2
            tz = dz * c0 + cz * c1 + vz * c2
            tx = jnp.minimum(jnp.maximum(tx, -100.0), 100.0)
            ty = jnp.minimum(jnp.maximum(ty, -100.0), 100.0)
            tz = jnp.minimum(jnp.maximum(tz, -100.0), 100.0)
            plsc.store_scatter(tc_v, [eidx, z], tx)
            plsc.store_scatter(tc_v, [eidx, z + 1], ty)
            plsc.store_scatter(tc_v, [eidx, z + 2], tz)
            plsc.store_scatter(tc_v, [eidx, z + 3], ones)

        pltpu.sync_copy(tc_v, tsc_sh.at[idxr_v], add=True)
        return carry
    lax.fori_loop(0, NCH, chunk, 0)

    plsc.subcore_barrier()
    pltpu.sync_copy(tsc_sh.at[rsl], tsc_out.at[cid, rsl])


def _stage_d2(row, col, coordp, coff, zts):
    fn = pl.kernel(
        _coordagg_body,
        mesh=_mesh,
        compiler_params=pltpu.CompilerParams(needs_layout_passes=False, use_tc_tiling_on_sc=False),
        out_type=jax.ShapeDtypeStruct((NC, N, TSW), _f32),
        scratch_types=[
            pltpu.VMEM((N, 4), _f32),
            pltpu.VMEM((CH,), jnp.int32),
            pltpu.VMEM((CH,), jnp.int32),
            pltpu.VMEM((CH, 8), _f32),
            pltpu.VMEM((CH, TSW), _f32),
            pltpu.VMEM_SHARED((N, TSW), _f32),
        ],
    )
    return fn(row, col, coordp, coff, zts)


def _node_body(h_ref, p0_ref, p1_ref, t0_ref, t1_ref, coord_ref,
               wnh_ref, wna_ref, nb1_ref, nw2_ref, nb2_ref, g_ref, b_ref,
               hout_ref, cout_ref):
    h = h_ref[...]
    agg = p0_ref[...] + p1_ref[...]
    t = jnp.dot(h, wnh_ref[...], preferred_element_type=_f32)
    t = t + jnp.dot(agg, wna_ref[...], preferred_element_type=_f32)
    t = jnp.maximum(t + nb1_ref[...], 0.0)
    nm = jnp.dot(t, nw2_ref[...], preferred_element_type=_f32) + nb2_ref[...]
    ho = 2.0 * h + nm
    mu = jnp.mean(ho, axis=1, keepdims=True)
    xc = ho - mu
    var = jnp.mean(xc * xc, axis=1, keepdims=True)
    hout_ref[...] = xc * lax.rsqrt(var + 1e-5) * g_ref[...] + b_ref[...]
    ts = t0_ref[...] + t1_ref[...]
    cnt = jnp.maximum(ts[:, 3:4], 1.0)
    cout_ref[...] = coord_ref[...] + ts / cnt


def _stage_e(h, p0, p1, t0, t1, coordp16, w_nh, w_na, nb1, nw2, nb2, g, b):
    bn = 2000
    grid = (N // bn,)
    return pl.pallas_call(
        _node_body,
        grid=grid,
        in_specs=[
            pl.BlockSpec((bn, D), lambda i: (i, 0)),
            pl.BlockSpec((bn, D), lambda i: (i, 0)),
            pl.BlockSpec((bn, D), lambda i: (i, 0)),
            pl.BlockSpec((bn, TSW), lambda i: (i, 0)),
            pl.BlockSpec((bn, TSW), lambda i: (i, 0)),
            pl.BlockSpec((bn, TSW), lambda i: (i, 0)),
            pl.BlockSpec((D, D), lambda i: (0, 0)),
            pl.BlockSpec((D, D), lambda i: (0, 0)),
            pl.BlockSpec((1, D), lambda i: (0, 0)),
            pl.BlockSpec((D, D), lambda i: (0, 0)),
            pl.BlockSpec((1, D), lambda i: (0, 0)),
            pl.BlockSpec((1, D), lambda i: (0, 0)),
            pl.BlockSpec((1, D), lambda i: (0, 0)),
        ],
        out_specs=[
            pl.BlockSpec((bn, D), lambda i: (i, 0)),
            pl.BlockSpec((bn, TSW), lambda i: (i, 0)),
        ],
        out_shape=[
            jax.ShapeDtypeStruct((N, D), _f32),
            jax.ShapeDtypeStruct((N, TSW), _f32),
        ],
    )(h, p0, p1, t0, t1, coordp16, w_nh, w_na, nb1, nw2, nb2, g, b)


def kernel(h, edge_index, coord, edge_attr,
           e_w1, e_b1, e_w2, e_b2,
           n_w1, n_b1, n_w2, n_b2,
           c_w1, c_b1, c_w2, ln_g, ln_b):
    row = edge_index[0]
    col = edge_index[1]
    w_hr = e_w1[0:D]
    w_hc = e_w1[D:2 * D]
    w_rad = e_w1[2 * D:2 * D + 1]
    w_attr = e_w1[2 * D + 1:]
    coordp = jnp.pad(coord, ((0, 0), (0, 1)))
    coordp16 = jnp.pad(coord, ((0, 0), (0, TSW - 3)))
    cw2p = jnp.pad(c_w2, ((0, 0), (0, 5)))
    z128 = jnp.zeros((N, D), _f32)
    zts = jnp.zeros((N, TSW), _f32)

    hr, hc = _stage_a(h, w_hr, w_hc, e_b1.reshape(1, D))
    pre, rad = _stage_b(hr, hc, row, col, coordp)
    ef, coff = _stage_c(pre, rad.reshape(E, 1), edge_attr,
                        w_rad, w_attr, e_w2, e_b2.reshape(1, D),
                        c_w1, c_b1.reshape(1, D), cw2p)
    aggh_p = _stage_d1(row, ef, z128)
    tsc_p = _stage_d2(row, col, coordp, coff, zts)
    h_out, coord16 = _stage_e(h, aggh_p[0], aggh_p[1], tsc_p[0], tsc_p[1],
                              coordp16, n_w1[0:D], n_w1[D:2 * D],
                              n_b1.reshape(1, D), n_w2, n_b2.reshape(1, D),
                              ln_g.reshape(1, D), ln_b.reshape(1, D))
    coord_out = coord16[:, 0:3]
    return (h_out, coord_out, edge_attr)


# speedup vs baseline: 5.3989x; 1.3686x over previous
import functools

import jax
import jax.numpy as jnp
from jax import lax
from jax.experimental import pallas as pl
from jax.experimental.pallas import tpu as pltpu
from jax.experimental.pallas import tpu_sc as plsc

N = 10000
E = 320000
D = 128
DE = 16
NC = 2
NS = 16
NW = NC * NS
EPW = E // NW
CH = 80
NCH = EPW // CH
GR = CH // 16
RPT = N // NS
TSW = 16

_mesh = plsc.VectorSubcoreMesh(core_axis_name="c", subcore_axis_name="s")
_f32 = jnp.float32


def _proj_body(h_ref, wr_ref, wc_ref, b1_ref, hr_ref, hc_ref):
    h = h_ref[...]
    hr_ref[...] = jnp.dot(h, wr_ref[...], preferred_element_type=_f32)
    hc_ref[...] = jnp.dot(h, wc_ref[...], preferred_element_type=_f32) + b1_ref[...]


def _stage_a(h, w_hr, w_hc, b1):
    bn = 2000
    grid = (N // bn,)
    return pl.pallas_call(
        _proj_body,
        grid=grid,
        in_specs=[
            pl.BlockSpec((bn, D), lambda i: (i, 0)),
            pl.BlockSpec((D, D), lambda i: (0, 0)),
            pl.BlockSpec((D, D), lambda i: (0, 0)),
            pl.BlockSpec((1, D), lambda i: (0, 0)),
        ],
        out_specs=[
            pl.BlockSpec((bn, D), lambda i: (i, 0)),
            pl.BlockSpec((bn, D), lambda i: (i, 0)),
        ],
        out_shape=[
            jax.ShapeDtypeStruct((N, D), _f32),
            jax.ShapeDtypeStruct((N, D), _f32),
        ],
    )(h, w_hr, w_hc, b1)


def _gather_body(hr_hbm, hc_hbm, row_hbm, col_hbm, coord_hbm,
                 pre_hbm, rad_hbm,
                 coord_v, idxr0, idxc0, idxr1, idxc1,
                 ra0, rb0, ra1, rb1, rad_v,
                 sa0, sb0, sa1, sb1):
    cid = lax.axis_index("c")
    sid = lax.axis_index("s")
    wid = sid * NC + cid
    base = wid * EPW
    pltpu.sync_copy(coord_hbm, coord_v)

    slots = ((idxr0, idxc0, ra0, rb0, sa0, sb0),
             (idxr1, idxc1, ra1, rb1, sa1, sb1))

    def fetch(c, slot):
        idxr, idxc, ra, rb, sa, sb = slots[slot]
        off = base + c * CH
        pltpu.sync_copy(row_hbm.at[pl.ds(off, CH)], idxr)
        pltpu.sync_copy(col_hbm.at[pl.ds(off, CH)], idxc)
        pltpu.async_copy(hr_hbm.at[idxr], ra, sa)
        pltpu.async_copy(hc_hbm.at[idxc], rb, sb)

    def process(c, slot):
        idxr, idxc, ra, rb, sa, sb = slots[slot]
        off = base + c * CH
        pltpu.make_async_copy(hr_hbm.at[idxr], ra, sa).wait()
        pltpu.make_async_copy(hc_hbm.at[idxc], rb, sb).wait()

        def addrow(i, acc):
            for j in range(D // 16):
                sl = pl.ds(j * 16, 16)
                plsc.addupdate(ra.at[i, sl], rb[i, sl])
            return acc
        lax.fori_loop(0, CH, addrow, 0)

        z = jnp.zeros((16,), jnp.int32)
        for g in range(GR):
            sl = pl.ds(g * 16, 16)
            rowv = idxr[sl]
            colv = idxc[sl]
            crx = plsc.load_gather(coord_v, [rowv, z])
            cry = plsc.load_gather(coord_v, [rowv, z + 1])
            crz = plsc.load_gather(coord_v, [rowv, z + 2])
            ccx = plsc.load_gather(coord_v, [colv, z])
            ccy = plsc.load_gather(coord_v, [colv, z + 1])
            ccz = plsc.load_gather(coord_v, [colv, z + 2])
            dx = crx - ccx
            dy = cry - ccy
            dz = crz - ccz
            rad_v[sl] = dx * dx + dy * dy + dz * dz

        pltpu.sync_copy(ra, pre_hbm.at[pl.ds(off, CH)])
        pltpu.sync_copy(rad_v, rad_hbm.at[pl.ds(off, CH)])

    fetch(0, 0)

    def pair(i, carry):
        c = 2 * i
        fetch(c + 1, 1)
        process(c, 0)
        fetch(c + 2, 0)
        process(c + 1, 1)
        return carry
    lax.fori_loop(0, (NCH - 1) // 2, pair, 0)
    process(NCH - 1, 0)


def _stage_b(hr, hc, row, col, coordp):
    fn = pl.kernel(
        _gather_body,
        mesh=_mesh,
        compiler_params=pltpu.CompilerParams(needs_layout_passes=False, use_tc_tiling_on_sc=False),
        out_type=[
            jax.ShapeDtypeStruct((E, D), _f32),
            jax.ShapeDtypeStruct((E,), _f32),
        ],
        scratch_types=[
            pltpu.VMEM((N, 4), _f32),
            pltpu.VMEM((CH,), jnp.int32),
            pltpu.VMEM((CH,), jnp.int32),
            pltpu.VMEM((CH,), jnp.int32),
            pltpu.VMEM((CH,), jnp.int32),
            pltpu.VMEM((CH, D), _f32),
            pltpu.VMEM((CH, D), _f32),
            pltpu.VMEM((CH, D), _f32),
            pltpu.VMEM((CH, D), _f32),
            pltpu.VMEM((CH,), _f32),
            pltpu.SemaphoreType.DMA,
            pltpu.SemaphoreType.DMA,
            pltpu.SemaphoreType.DMA,
            pltpu.SemaphoreType.DMA,
        ],
    )
    return fn(hr, hc, row, col, coordp)


def _edge_mlp_body(pre_ref, rad_ref, attr_ref, wrad_ref, wattr_ref,
                   w2_ref, b2_ref, cw1_ref, cb1_ref, cw2_ref,
                   ef_ref, coff_ref):
    x = pre_ref[...] + rad_ref[...] * wrad_ref[...]
    x = x + jnp.dot(attr_ref[...], wattr_ref[...], preferred_element_type=_f32)
    x = jnp.maximum(x, 0.0)
    ef = jnp.dot(x, w2_ref[...], preferred_element_type=_f32) + b2_ref[...]
    ef = jnp.maximum(ef, 0.0)
    y = jnp.dot(ef, cw1_ref[...], preferred_element_type=_f32) + cb1_ref[...]
    y = jnp.maximum(y, 0.0)
    coff_ref[...] = jnp.dot(y, cw2_ref[...], preferred_element_type=_f32)
    ef_ref[...] = ef


def _stage_c(pre, rad, attr, w_rad, w_attr, e_w2, e_b2, c_w1, c_b1, cw2p):
    be = 2000
    grid = (E // be,)
    return pl.pallas_call(
        _edge_mlp_body,
        grid=grid,
        in_specs=[
            pl.BlockSpec((be, D), lambda i: (i, 0)),
            pl.BlockSpec((be, 1), lambda i: (i, 0)),
            pl.BlockSpec((be, DE), lambda i: (i, 0)),
            pl.BlockSpec((1, D), lambda i: (0, 0)),
            pl.BlockSpec((DE, D), lambda i: (0, 0)),
            pl.BlockSpec((D, D), lambda i: (0, 0)),
            pl.BlockSpec((1, D), lambda i: (0, 0)),
            pl.BlockSpec((D, D), lambda i: (0, 0)),
            pl.BlockSpec((1, D), lambda i: (0, 0)),
            pl.BlockSpec((D, 8), lambda i: (0, 0)),
        ],
        out_specs=[
            pl.BlockSpec((be, D), lambda i: (i, 0)),
            pl.BlockSpec((be, 8), lambda i: (i, 0)),
        ],
        out_shape=[
            jax.ShapeDtypeStruct((E, D), _f32),
            jax.ShapeDtypeStruct((E, 8), _f32),
        ],
    )(pre, rad, attr, w_rad, w_attr, e_w2, e_b2, c_w1, c_b1, cw2p)


def _aggh_body(row_hbm, ef_hbm, z128_hbm, aggh_out,
               idx0, idx1, ef0, ef1, aggh_sh, se0, se1):
    cid = lax.axis_index("c")
    sid = lax.axis_index("s")
    wid = sid * NC + cid
    base = wid * EPW

    rsl = pl.ds(sid * RPT, RPT)
    pltpu.sync_copy(z128_hbm.at[rsl], aggh_sh.at[rsl])
    plsc.subcore_barrier()

    slots = ((idx0, ef0, se0), (idx1, ef1, se1))

    def fetch(c, slot):
        idx, ef, se = slots[slot]
        off = base + c * CH
        pltpu.sync_copy(row_hbm.at[pl.ds(off, CH)], idx)
        pltpu.async_copy(ef_hbm.at[pl.ds(off, CH)], ef, se)

    def process(c, slot):
        idx, ef, se = slots[slot]
        off = base + c * CH
        pltpu.make_async_copy(ef_hbm.at[pl.ds(off, CH)], ef, se).wait()
        pltpu.sync_copy(ef, aggh_sh.at[idx], add=True)

    fetch(0, 0)

    def pair(i, carry):
        c = 2 * i
        fetch(c + 1, 1)
        process(c, 0)
        fetch(c + 2, 0)
        process(c + 1, 1)
        return carry
    lax.fori_loop(0, (NCH - 1) // 2, pair, 0)
    process(NCH - 1, 0)

    plsc.subcore_barrier()
    pltpu.sync_copy(aggh_sh.at[rsl], aggh_out.at[cid, rsl])


def _stage_d1(row, ef, z128):
    fn = pl.kernel(
        _aggh_body,
        mesh=_mesh,
        compiler_params=pltpu.CompilerParams(needs_layout_passes=False, use_tc_tiling_on_sc=False),
        out_type=jax.ShapeDtypeStruct((NC, N, D), _f32),
        scratch_types=[
            pltpu.VMEM((CH,), jnp.int32),
            pltpu.VMEM((CH,), jnp.int32),
            pltpu.VMEM((CH, D), _f32),
            pltpu.VMEM((CH, D), _f32),
            pltpu.VMEM_SHARED((N, D), _f32),
            pltpu.SemaphoreType.DMA,
            pltpu.SemaphoreType.DMA,
        ],
    )
    return fn(row, ef, z128)


def _coordagg_body(row_hbm, col_hbm, coord_hbm, coff_hbm, zts_hbm,
                   tsc_out,
                   coord_v, idxr0, idxc0, coff0, idxr1, idxc1, coff1,
                   tc_v, tsc_sh, sc0, sc1):
    cid = lax.axis_index("c")
    sid = lax.axis_index("s")
    wid = sid * NC + cid
    base = wid * EPW
    pltpu.sync_copy(coord_hbm, coord_v)

    def ztc(i, acc):
        tc_v[i, pl.ds(0, TSW)] = jnp.zeros((TSW,), _f32)
        return acc
    lax.fori_loop(0, CH, ztc, 0)

    rsl = pl.ds(sid * RPT, RPT)
    pltpu.sync_copy(zts_hbm.at[rsl], tsc_sh.at[rsl])
    plsc.subcore_barrier()

    iota = lax.iota(jnp.int32, 16)
    ones = jnp.ones((16,), _f32)
    slots = ((idxr0, idxc0, coff0, sc0), (idxr1, idxc1, coff1, sc1))

    def fetch(c, slot):
        idxr_v, idxc_v, coff_v, sc = slots[slot]
        off = base + c * CH
        pltpu.sync_copy(row_hbm.at[pl.ds(off, CH)], idxr_v)
        pltpu.sync_copy(col_hbm.at[pl.ds(off, CH)], idxc_v)
        pltpu.async_copy(coff_hbm.at[pl.ds(off, CH)], coff_v, sc)

    def process(c, slot):
        idxr_v, idxc_v, coff_v, sc = slots[slot]
        off = base + c * CH
        pltpu.make_async_copy(coff_hbm.at[pl.ds(off, CH)], coff_v, sc).wait()

        z = jnp.zeros((16,), jnp.int32)
        for g in range(GR):
            sl = pl.ds(g * 16, 16)
            rowv = idxr_v[sl]
            colv = idxc_v[sl]
            eidx = iota + (g * 16)
            crx = plsc.load_gather(coord_v, [rowv, z])
            cry = plsc.load_gather(coord_v, [rowv, z + 1])
            crz = plsc.load_gather(coord_v, [rowv, z + 2])
            ccx = plsc.load_gather(coord_v, [colv, z])
            ccy = plsc.load_gather(coord_v, [colv, z + 1])
            ccz = plsc.load_gather(coord_v, [colv, z + 2])
            c0 = plsc.load_gather(coff_v, [eidx, z])
            c1 = plsc.load_gather(coff_v, [eidx, z + 1])
            c2 = plsc.load_gather(coff_v, [eidx, z + 2])
            dx = crx - ccx
            dy = cry - ccy
            dz = crz - ccz
            cx = cry * ccz - crz * ccy
            cy = crz * ccx - crx * ccz
            cz = crx * ccy - cry * ccx
            vx = dy * cz - dz * cy
            vy = dz * cx - dx * cz
            vz = dx * cy - dy * cx
            tx = dx * c0 + cx * c1 + vx * c2
            ty = dy * c0 + cy * c1 + vy * c2
            tz = dz * c0 + cz * c1 + vz * c2
            tx = jnp.minimum(jnp.maximum(tx, -100.0), 100.0)
            ty = jnp.minimum(jnp.maximum(ty, -100.0), 100.0)
            tz = jnp.minimum(jnp.maximum(tz, -100.0), 100.0)
            plsc.store_scatter(tc_v, [eidx, z], tx)
            plsc.store_scatter(tc_v, [eidx, z + 1], ty)
            plsc.store_scatter(tc_v, [eidx, z + 2], tz)
            plsc.store_scatter(tc_v, [eidx, z + 3], ones)

        pltpu.sync_copy(tc_v, tsc_sh.at[idxr_v], add=True)

    fetch(0, 0)

    def pair(i, carry):
        c = 2 * i
        fetch(c + 1, 1)
        process(c, 0)
        fetch(c + 2, 0)
        process(c + 1, 1)
        return carry
    lax.fori_loop(0, (NCH - 1) // 2, pair, 0)
    process(NCH - 1, 0)

    plsc.subcore_barrier()
    pltpu.sync_copy(tsc_sh.at[rsl], tsc_out.at[cid, rsl])


def _stage_d2(row, col, coordp, coff, zts):
    fn = pl.kernel(
        _coordagg_body,
        mesh=_mesh,
        compiler_params=pltpu.CompilerParams(needs_layout_passes=False, use_tc_tiling_on_sc=False),
        out_type=jax.ShapeDtypeStruct((NC, N, TSW), _f32),
        scratch_types=[
            pltpu.VMEM((N, 4), _f32),
            pltpu.VMEM((CH,), jnp.int32),
            pltpu.VMEM((CH,), jnp.int32),
            pltpu.VMEM((CH, 8), _f32),
            pltpu.VMEM((CH,), jnp.int32),
            pltpu.VMEM((CH,), jnp.int32),
            pltpu.VMEM((CH, 8), _f32),
            pltpu.VMEM((CH, TSW), _f32),
            pltpu.VMEM_SHARED((N, TSW), _f32),
            pltpu.SemaphoreType.DMA,
            pltpu.SemaphoreType.DMA,
        ],
    )
    return fn(row, col, coordp, coff, zts)


def _node_body(h_ref, p0_ref, p1_ref, t0_ref, t1_ref, coord_ref,
               wnh_ref, wna_ref, nb1_ref, nw2_ref, nb2_ref, g_ref, b_ref,
               hout_ref, cout_ref):
    h = h_ref[...]
    agg = p0_ref[...] + p1_ref[...]
    t = jnp.dot(h, wnh_ref[...], preferred_element_type=_f32)
    t = t + jnp.dot(agg, wna_ref[...], preferred_element_type=_f32)
    t = jnp.maximum(t + nb1_ref[...], 0.0)
    nm = jnp.dot(t, nw2_ref[...], preferred_element_type=_f32) + nb2_ref[...]
    ho = 2.0 * h + nm
    mu = jnp.mean(ho, axis=1, keepdims=True)
    xc = ho - mu
    var = jnp.mean(xc * xc, axis=1, keepdims=True)
    hout_ref[...] = xc * lax.rsqrt(var + 1e-5) * g_ref[...] + b_ref[...]
    ts = t0_ref[...] + t1_ref[...]
    cnt = jnp.maximum(ts[:, 3:4], 1.0)
    cout_ref[...] = coord_ref[...] + ts / cnt


def _stage_e(h, p0, p1, t0, t1, coordp16, w_nh, w_na, nb1, nw2, nb2, g, b):
    bn = 2000
    grid = (N // bn,)
    return pl.pallas_call(
        _node_body,
        grid=grid,
        in_specs=[
            pl.BlockSpec((bn, D), lambda i: (i, 0)),
            pl.BlockSpec((bn, D), lambda i: (i, 0)),
            pl.BlockSpec((bn, D), lambda i: (i, 0)),
            pl.BlockSpec((bn, TSW), lambda i: (i, 0)),
            pl.BlockSpec((bn, TSW), lambda i: (i, 0)),
            pl.BlockSpec((bn, TSW), lambda i: (i, 0)),
            pl.BlockSpec((D, D), lambda i: (0, 0)),
            pl.BlockSpec((D, D), lambda i: (0, 0)),
            pl.BlockSpec((1, D), lambda i: (0, 0)),
            pl.BlockSpec((D, D), lambda i: (0, 0)),
            pl.BlockSpec((1, D), lambda i: (0, 0)),
            pl.BlockSpec((1, D), lambda i: (0, 0)),
            pl.BlockSpec((1, D), lambda i: (0, 0)),
        ],
        out_specs=[
            pl.BlockSpec((bn, D), lambda i: (i, 0)),
            pl.BlockSpec((bn, TSW), lambda i: (i, 0)),
        ],
        out_shape=[
            jax.ShapeDtypeStruct((N, D), _f32),
            jax.ShapeDtypeStruct((N, TSW), _f32),
        ],
    )(h, p0, p1, t0, t1, coordp16, w_nh, w_na, nb1, nw2, nb2, g, b)


def kernel(h, edge_index, coord, edge_attr,
           e_w1, e_b1, e_w2, e_b2,
           n_w1, n_b1, n_w2, n_b2,
           c_w1, c_b1, c_w2, ln_g, ln_b):
    row = edge_index[0]
    col = edge_index[1]
    w_hr = e_w1[0:D]
    w_hc = e_w1[D:2 * D]
    w_rad = e_w1[2 * D:2 * D + 1]
    w_attr = e_w1[2 * D + 1:]
    coordp = jnp.pad(coord, ((0, 0), (0, 1)))
    coordp16 = jnp.pad(coord, ((0, 0), (0, TSW - 3)))
    cw2p = jnp.pad(c_w2, ((0, 0), (0, 5)))
    z128 = jnp.zeros((N, D), _f32)
    zts = jnp.zeros((N, TSW), _f32)

    hr, hc = _stage_a(h, w_hr, w_hc, e_b1.reshape(1, D))
    pre, rad = _stage_b(hr, hc, row, col, coordp)
    ef, coff = _stage_c(pre, rad.reshape(E, 1), edge_attr,
                        w_rad, w_attr, e_w2, e_b2.reshape(1, D),
                        c_w1, c_b1.reshape(1, D), cw2p)
    aggh_p = _stage_d1(row, ef, z128)
    tsc_p = _stage_d2(row, col, coordp, coff, zts)
    h_out, coord16 = _stage_e(h, aggh_p[0], aggh_p[1], tsc_p[0], tsc_p[1],
                              coordp16, n_w1[0:D], n_w1[D:2 * D],
                              n_b1.reshape(1, D), n_w2, n_b2.reshape(1, D),
                              ln_g.reshape(1, D), ln_b.reshape(1, D))
    coord_out = coord16[:, 0:3]
    return (h_out, coord_out, edge_attr)
